# Initial kernel scaffold; baseline (speedup 1.0000x reference)
import jax
import jax.numpy as jnp
from jax.experimental import pallas as pl


def kernel(attn_fts, rel_edges, W_embed, b_embed, rel_weight_0, loop_weight_0, rel_weight_1, loop_weight_1):
    raise NotImplementedError("write your pallas kernel here")


# speedup vs baseline: 1.1608x; 1.1608x over previous
import jax
import jax.numpy as jnp
from jax.experimental import pallas as pl

_B, _N, _R = 32, 256, 16
_D_IN, _D_H = 128, 128
_PREC = jax.lax.Precision.DEFAULT


def _rgcn_body(attn_ref, A_ref, We_ref, be_ref,
               Wr0_ref, Lw0_ref, Wr1_ref, Lw1_ref, out_ref):
    f32 = jnp.float32
    x = jnp.dot(attn_ref[0], We_ref[...], preferred_element_type=f32, precision=_PREC)
    x = jnp.maximum(x + be_ref[...], 0.0)
    A2 = A_ref[0].reshape(_R * _N, _N)
    for Wr_ref, Lw_ref in ((Wr0_ref, Lw0_ref), (Wr1_ref, Lw1_ref)):
        nb = jnp.dot(A2, x, preferred_element_type=f32, precision=_PREC)
        msg = jax.lax.dot_general(
            nb.reshape(_R, _N, _D_H), Wr_ref[...],
            dimension_numbers=(((2,), (1,)), ((0,), (0,))),
            preferred_element_type=f32, precision=_PREC)
        acc = jnp.dot(x, Lw_ref[...], preferred_element_type=f32, precision=_PREC)
        x = jnp.maximum(acc + msg.sum(axis=0), 0.0)
    out_ref[0] = x


def kernel(attn_fts, rel_edges, W_embed, b_embed,
           rel_weight_0, loop_weight_0, rel_weight_1, loop_weight_1):
    B, N, D_IN = attn_fts.shape
    R = rel_edges.shape[1]
    D_H = W_embed.shape[1]
    b2 = b_embed.reshape(1, D_H)
    grid = (B,)
    return pl.pallas_call(
        _rgcn_body,
        grid=grid,
        in_specs=[
            pl.BlockSpec((1, N, D_IN), lambda b: (b, 0, 0)),
            pl.BlockSpec((1, R, N, N), lambda b: (b, 0, 0, 0)),
            pl.BlockSpec((D_IN, D_H), lambda b: (0, 0)),
            pl.BlockSpec((1, D_H), lambda b: (0, 0)),
            pl.BlockSpec((R, D_H, D_H), lambda b: (0, 0, 0)),
            pl.BlockSpec((D_H, D_H), lambda b: (0, 0)),
            pl.BlockSpec((R, D_H, D_H), lambda b: (0, 0, 0)),
            pl.BlockSpec((D_H, D_H), lambda b: (0, 0)),
        ],
        out_specs=pl.BlockSpec((1, N, D_H), lambda b: (b, 0, 0)),
        out_shape=jax.ShapeDtypeStruct((B, N, D_H), jnp.float32),
    )(attn_fts, rel_edges, W_embed, b2,
      rel_weight_0, loop_weight_0, rel_weight_1, loop_weight_1)
